# baseline (speedup 1.0000x reference)
import jax
import jax.numpy as jnp
from jax.experimental import pallas as pl
from jax.experimental.pallas import tpu as pltpu

_NUM_GRAPHS = 100
_K = 6
_D = 128
_GPB = 10


def _prelu(v, a):
    return jnp.where(v >= 0, v, a * v)


def _graph_kernel(xref,
                  pW1, pb1, pa1, pW2, pb2, pa2,
                  cW0, cb0, cW1, cb1, cW2, cb2, cW3, cb3,
                  qW1, qb1, qa1, qW2, qb2, qa2,
                  oref):
    for gi in range(_GPB):
        _one_graph(xref, gi,
                   pW1, pb1, pa1, pW2, pb2, pa2,
                   cW0, cb0, cW1, cb1, cW2, cb2, cW3, cb3,
                   qW1, qb1, qa1, qW2, qb2, qa2, oref)


def _one_graph(xref, gi,
               pW1, pb1, pa1, pW2, pb2, pa2,
               cW0, cb0, cW1, cb1, cW2, cb2, cW3, cb3,
               qW1, qb1, qa1, qW2, qb2, qa2, oref):
    xb = xref[gi]
    n = xb.shape[0]
    f32 = jnp.float32
    npad = (-n) % 8

    m = n + npad
    xp = jnp.concatenate([xb, jnp.full((npad, _D), 1e4, f32)], axis=0)
    x2 = xp * xp
    ones = jnp.ones((1, _D), f32)
    sq = jax.lax.dot_general(x2, ones, (((1,), (1,)), ((), ())),
                             preferred_element_type=f32)
    gram = jax.lax.dot_general(xp, xp, (((1,), (1,)), ((), ())),
                               preferred_element_type=f32)
    col = jax.lax.broadcasted_iota(jnp.int32, (m, m), 1)
    row = jax.lax.broadcasted_iota(jnp.int32, (m, m), 0)
    y = sq - 2.0 * gram
    y = jnp.where(row == col, f32(1e10), y)

    regs = [jnp.full((8, m), 1e30, f32) for _ in range(_K)]
    for g in range(m // 8):
        s = y[8 * g:8 * g + 8, :]
        for t in range(_K):
            keep = jnp.minimum(regs[t], s)
            s = jnp.maximum(regs[t], s)
            regs[t] = keep
    cand = jnp.concatenate(regs, axis=0)
    for _ in range(_K - 1):
        m = jnp.min(cand, axis=0, keepdims=True)
        cand = jnp.where(cand == m, f32(1e30), cand)
    thr = jnp.min(cand, axis=0, keepdims=True)

    A = jnp.transpose((y <= thr).astype(f32))

    h = _prelu(jnp.dot(xp, pW1[...], preferred_element_type=f32) + pb1[0],
               pa1[0])
    h = _prelu(jnp.dot(h, pW2[...], preferred_element_type=f32) + pb2[0],
               pa2[0])

    for Wm, bm in ((cW0, cb0), (cW1, cb1), (cW2, cb2), (cW3, cb3)):
        msg = jnp.dot(h, Wm[...], preferred_element_type=f32) + bm[0]
        h = jnp.dot(A, msg, preferred_element_type=f32) + h

    h = _prelu(jnp.dot(h, qW1[...], preferred_element_type=f32) + qb1[0],
               qa1[0])
    h = _prelu(jnp.dot(h, qW2[...], preferred_element_type=f32) + qb2[0],
               qa2[0])

    oref[gi] = jnp.sum(h[:n], axis=0, keepdims=True)


def _head_kernel(pref, W1, b1, W2, b2, W3, b3, W4, b4, oref):
    p = pref[...]
    z = jnp.dot(p, W1[...], preferred_element_type=jnp.float32) + b1[0]
    z = jnp.where(z >= 0, z, 0.2 * z)
    z = jnp.dot(z, W2[...], preferred_element_type=jnp.float32) + b2[0]
    z = jnp.where(z >= 0, z, 0.2 * z)
    z = jnp.dot(z, W3[...], preferred_element_type=jnp.float32) + b3[0]
    z = jnp.where(z >= 0, z, 0.2 * z)
    z = jnp.dot(z, W4[...], preferred_element_type=jnp.float32) + b4[0]
    oref[...] = z


def kernel(x, batch_ids, params):
    n_total, d = x.shape
    nper = n_total // _NUM_GRAPHS
    xg = x.reshape(_NUM_GRAPHS, nper, d)

    def vec(name):
        return params[name].reshape(1, -1)

    weights = [
        params['pre_W1'], vec('pre_b1'), vec('pre_a1'),
        params['pre_W2'], vec('pre_b2'), vec('pre_a2'),
        params['conv0_Wm'], vec('conv0_bm'),
        params['conv1_Wm'], vec('conv1_bm'),
        params['conv2_Wm'], vec('conv2_bm'),
        params['conv3_Wm'], vec('conv3_bm'),
        params['post_W1'], vec('post_b1'), vec('post_a1'),
        params['post_W2'], vec('post_b2'), vec('post_a2'),
    ]

    in_specs = [pl.BlockSpec((_GPB, nper, d), lambda i: (i, 0, 0))]
    for w in weights:
        in_specs.append(pl.BlockSpec(w.shape, lambda i: (0, 0)))

    pooled = pl.pallas_call(
        _graph_kernel,
        grid=(_NUM_GRAPHS // _GPB,),
        in_specs=in_specs,
        out_specs=pl.BlockSpec((_GPB, 1, d), lambda i: (i, 0, 0)),
        out_shape=jax.ShapeDtypeStruct((_NUM_GRAPHS, 1, d), jnp.float32),
        compiler_params=pltpu.CompilerParams(
            dimension_semantics=("parallel",)),
    )(xg, *weights)

    pooled = pooled.reshape(_NUM_GRAPHS, d)

    z = pl.pallas_call(
        _head_kernel,
        out_shape=jax.ShapeDtypeStruct((_NUM_GRAPHS, 1), jnp.float32),
    )(pooled,
      params['hlv_W1'], vec('hlv_b1'),
      params['hlv_W2'], vec('hlv_b2'),
      params['hlv_W3'], vec('hlv_b3'),
      params['hlv_W4'], vec('hlv_b4'))
    return z


# speedup vs baseline: 1.0437x; 1.0437x over previous
import jax
import jax.numpy as jnp
from jax.experimental import pallas as pl
from jax.experimental.pallas import tpu as pltpu

_NUM_GRAPHS = 100
_K = 6
_D = 128
_GPB = 10


def _prelu(v, a):
    return jnp.where(v >= 0, v, a * v)


def _cswap(a, b):
    return jnp.minimum(a, b), jnp.maximum(a, b)


def _merge22(a, b):
    c0, t = _cswap(a[0], b[0])
    s, c3 = _cswap(a[1], b[1])
    c1, c2 = _cswap(t, s)
    return [c0, c1, c2, c3]


def _merge44(a, b):
    L = [jnp.minimum(a[i], b[3 - i]) for i in range(4)]
    H = [jnp.maximum(a[i], b[3 - i]) for i in range(4)]
    p0, p2 = _cswap(L[0], L[2])
    p1, p3 = _cswap(L[1], L[3])
    q0, q1 = _cswap(p0, p1)
    q2, q3 = _cswap(p2, p3)
    u0 = jnp.minimum(H[0], H[2])
    u1 = jnp.minimum(H[1], H[3])
    u0, u1 = _cswap(u0, u1)
    return [q0, q1, q2, q3, u0, u1]


def _merge66(a, b, final):
    mm = [jnp.minimum(a[i], b[5 - i]) for i in range(6)]
    if final:
        return mm
    mm[0], mm[3] = _cswap(mm[0], mm[3])
    mm[1], mm[4] = _cswap(mm[1], mm[4])
    mm[2], mm[5] = _cswap(mm[2], mm[5])
    for base in (0, 3):
        mm[base], mm[base + 2] = _cswap(mm[base], mm[base + 2])
        mm[base], mm[base + 1] = _cswap(mm[base], mm[base + 1])
        mm[base + 1], mm[base + 2] = _cswap(mm[base + 1], mm[base + 2])
    return mm


def _graph_kernel(xref,
                  pW1, pb1, pa1, pW2, pb2, pa2,
                  cW0, cb0, cW1, cb1, cW2, cb2, cW3, cb3,
                  qW1, qb1, qa1, qW2, qb2, qa2,
                  oref):
    for gi in range(_GPB):
        _one_graph(xref, gi,
                   pW1, pb1, pa1, pW2, pb2, pa2,
                   cW0, cb0, cW1, cb1, cW2, cb2, cW3, cb3,
                   qW1, qb1, qa1, qW2, qb2, qa2, oref)


def _one_graph(xref, gi,
               pW1, pb1, pa1, pW2, pb2, pa2,
               cW0, cb0, cW1, cb1, cW2, cb2, cW3, cb3,
               qW1, qb1, qa1, qW2, qb2, qa2, oref):
    xb = xref[gi]
    n = xb.shape[0]
    f32 = jnp.float32
    npad = (-n) % 8

    m = n + npad
    xp = jnp.concatenate([xb, jnp.full((npad, _D), 1e4, f32)], axis=0)
    x2 = xp * xp
    ones = jnp.ones((1, _D), f32)
    sq = jax.lax.dot_general(x2, ones, (((1,), (1,)), ((), ())),
                             preferred_element_type=f32)
    gram = jax.lax.dot_general(xp, xp, (((1,), (1,)), ((), ())),
                               preferred_element_type=f32)
    col = jax.lax.broadcasted_iota(jnp.int32, (m, m), 1)
    row = jax.lax.broadcasted_iota(jnp.int32, (m, m), 0)
    y = sq - 2.0 * gram
    y = jnp.where(row == col, f32(1e10), y)

    slabs = [y[8 * g:8 * g + 8, :] for g in range(m // 8)]
    if len(slabs) % 2:
        slabs.append(jnp.full((8, m), 1e30, f32))
    pairs = [_cswap(slabs[i], slabs[i + 1])
             for i in range(0, len(slabs), 2)]
    fours = [_merge22(pairs[i], pairs[i + 1])
             for i in range(0, len(pairs), 2)]
    sixes = [_merge44(fours[i], fours[i + 1])
             for i in range(0, len(fours), 2)]
    while len(sixes) > 1:
        nxt = [_merge66(sixes[i], sixes[i + 1], final=(len(sixes) == 2))
               for i in range(0, len(sixes), 2)]
        sixes = nxt
    cand = jnp.concatenate(sixes[0], axis=0)
    for _ in range(_K - 1):
        m = jnp.min(cand, axis=0, keepdims=True)
        cand = jnp.where(cand == m, f32(1e30), cand)
    thr = jnp.min(cand, axis=0, keepdims=True)

    A = jnp.transpose((y <= thr).astype(f32))

    h = _prelu(jnp.dot(xp, pW1[...], preferred_element_type=f32) + pb1[0],
               pa1[0])
    h = _prelu(jnp.dot(h, pW2[...], preferred_element_type=f32) + pb2[0],
               pa2[0])

    for Wm, bm in ((cW0, cb0), (cW1, cb1), (cW2, cb2), (cW3, cb3)):
        msg = jnp.dot(h, Wm[...], preferred_element_type=f32) + bm[0]
        h = jnp.dot(A, msg, preferred_element_type=f32) + h

    h = _prelu(jnp.dot(h, qW1[...], preferred_element_type=f32) + qb1[0],
               qa1[0])
    h = _prelu(jnp.dot(h, qW2[...], preferred_element_type=f32) + qb2[0],
               qa2[0])

    oref[gi] = jnp.sum(h[:n], axis=0, keepdims=True)


def _head_kernel(pref, W1, b1, W2, b2, W3, b3, W4, b4, oref):
    p = pref[...]
    z = jnp.dot(p, W1[...], preferred_element_type=jnp.float32) + b1[0]
    z = jnp.where(z >= 0, z, 0.2 * z)
    z = jnp.dot(z, W2[...], preferred_element_type=jnp.float32) + b2[0]
    z = jnp.where(z >= 0, z, 0.2 * z)
    z = jnp.dot(z, W3[...], preferred_element_type=jnp.float32) + b3[0]
    z = jnp.where(z >= 0, z, 0.2 * z)
    z = jnp.dot(z, W4[...], preferred_element_type=jnp.float32) + b4[0]
    oref[...] = z


def kernel(x, batch_ids, params):
    n_total, d = x.shape
    nper = n_total // _NUM_GRAPHS
    xg = x.reshape(_NUM_GRAPHS, nper, d)

    def vec(name):
        return params[name].reshape(1, -1)

    weights = [
        params['pre_W1'], vec('pre_b1'), vec('pre_a1'),
        params['pre_W2'], vec('pre_b2'), vec('pre_a2'),
        params['conv0_Wm'], vec('conv0_bm'),
        params['conv1_Wm'], vec('conv1_bm'),
        params['conv2_Wm'], vec('conv2_bm'),
        params['conv3_Wm'], vec('conv3_bm'),
        params['post_W1'], vec('post_b1'), vec('post_a1'),
        params['post_W2'], vec('post_b2'), vec('post_a2'),
    ]

    in_specs = [pl.BlockSpec((_GPB, nper, d), lambda i: (i, 0, 0))]
    for w in weights:
        in_specs.append(pl.BlockSpec(w.shape, lambda i: (0, 0)))

    pooled = pl.pallas_call(
        _graph_kernel,
        grid=(_NUM_GRAPHS // _GPB,),
        in_specs=in_specs,
        out_specs=pl.BlockSpec((_GPB, 1, d), lambda i: (i, 0, 0)),
        out_shape=jax.ShapeDtypeStruct((_NUM_GRAPHS, 1, d), jnp.float32),
        compiler_params=pltpu.CompilerParams(
            dimension_semantics=("parallel",)),
    )(xg, *weights)

    pooled = pooled.reshape(_NUM_GRAPHS, d)

    z = pl.pallas_call(
        _head_kernel,
        out_shape=jax.ShapeDtypeStruct((_NUM_GRAPHS, 1), jnp.float32),
    )(pooled,
      params['hlv_W1'], vec('hlv_b1'),
      params['hlv_W2'], vec('hlv_b2'),
      params['hlv_W3'], vec('hlv_b3'),
      params['hlv_W4'], vec('hlv_b4'))
    return z
